# baseline (speedup 1.0000x reference)
import math

import jax
import jax.numpy as jnp
from jax import lax
from jax.experimental import pallas as pl
from jax.experimental.pallas import tpu as pltpu
from jax.experimental.pallas import tpu_sc as plsc

D_MODEL = 64
SCALE = math.sqrt(D_MODEL)

NC = 2
NS = 16
NW = NC * NS

NBUF = 8
LOOK = 4
DRAIN = NBUF - LOOK
GROUP = NBUF


def _emb_body(x_hbm, lut_hbm, out_hbm, idx_v, rows_v, gsem, ssem, *,
              rows_per_w, seq):
    wid = lax.axis_index("s") * NC + lax.axis_index("c")
    n_groups = rows_per_w // GROUP
    row_base = wid * rows_per_w

    def load_idx(k):
        pltpu.sync_copy(x_hbm.at[pl.ds(row_base + k * GROUP, GROUP)],
                        idx_v.at[k % 2])

    def fire_gather(slot, r, b):
        pltpu.async_copy(lut_hbm.at[idx_v.at[slot, r]], rows_v.at[b], gsem)

    def wait_gather(b):
        pltpu.make_async_copy(lut_hbm.at[idx_v.at[0, 0]], rows_v.at[b],
                              gsem).wait()

    def scale(b):
        @plsc.parallel_loop(0, seq, 1, unroll=4)
        def _(r):
            for k in range(D_MODEL // 16):
                rows_v[b, r, pl.ds(k * 16, 16)] = (
                    rows_v[b, r, pl.ds(k * 16, 16)] * SCALE)

    def fire_scatter(s, b):
        pltpu.async_copy(rows_v.at[b], out_hbm.at[row_base + s], ssem)

    def wait_scatter():
        pltpu.make_async_copy(rows_v.at[0], out_hbm.at[row_base], ssem).wait()

    load_idx(0)
    for b in range(LOOK):
        fire_gather(0, b, b)

    def group_body(g, *, first, last):
        for b in range(GROUP):
            s = g * GROUP + b
            if b == LOOK and not last:
                load_idx(g + 1)
            wait_gather(b)
            scale(b)
            fire_scatter(s, b)
            if not (first and b < DRAIN):
                wait_scatter()
            if not (last and b >= GROUP - LOOK):
                slot = (g + (1 if b >= GROUP - LOOK else 0)) % 2
                fire_gather(slot, (b + LOOK) % GROUP, (b + LOOK) % NBUF)

    group_body(0, first=True, last=False)

    def mid(g, carry):
        group_body(g, first=False, last=False)
        return carry
    lax.fori_loop(1, n_groups - 1, mid, 0)

    group_body(n_groups - 1, first=False, last=True)

    for _ in range(DRAIN):
        wait_scatter()


@jax.jit
def _run(x, lut):
    batch, seq = x.shape
    rows_per_w = batch // NW
    mesh = plsc.VectorSubcoreMesh(core_axis_name="c", subcore_axis_name="s",
                                  num_cores=NC, num_subcores=NS)

    def body(x_ref, lut_ref, out_ref, idx_v, rows_v, gsem, ssem):
        _emb_body(x_ref, lut_ref, out_ref, idx_v, rows_v, gsem, ssem,
                  rows_per_w=rows_per_w, seq=seq)

    f = pl.kernel(
        body,
        out_type=jax.ShapeDtypeStruct((batch, seq, D_MODEL), jnp.float32),
        mesh=mesh,
        scratch_types=[
            pltpu.VMEM((2, GROUP, seq), jnp.int32),
            pltpu.VMEM((NBUF, seq, D_MODEL), jnp.float32),
            pltpu.SemaphoreType.DMA,
            pltpu.SemaphoreType.DMA,
        ],
        compiler_params=pltpu.CompilerParams(use_tc_tiling_on_sc=False),
    )
    return f(x, lut)


def kernel(x, lut):
    assert x.shape[0] % (NW * GROUP) == 0
    return _run(x, lut)
